# Initial kernel scaffold; baseline (speedup 1.0000x reference)
import jax
import jax.numpy as jnp
from jax.experimental import pallas as pl


def kernel(x, W1, W2):
    raise NotImplementedError("write your pallas kernel here")


# speedup vs baseline: 4.9417x; 4.9417x over previous
import functools

import jax
import jax.numpy as jnp
from jax import lax
from jax.experimental import pallas as pl
from jax.experimental.pallas import tpu as pltpu
from jax.experimental.pallas import tpu_sc as plsc

K = 20
ROWS = 256
R3 = 256
GW = 128


def _knn_body(xfull_ref, xtile_ref, w1at_ref, wdt_ref, idx_ref, a_ref, bv_ref):
    xb = xfull_ref[0]
    xi = xtile_ref[0]
    n = xb.shape[1]
    rows = xi.shape[1]

    dn = (((0,), (0,)), ((), ()))
    a_ref[0] = lax.dot_general(xi, w1at_ref[...], dn,
                               preferred_element_type=jnp.float32)
    bv_ref[0] = lax.dot_general(xi, wdt_ref[...], dn,
                                preferred_element_type=jnp.float32)

    inner = lax.dot_general(xi, xb, dn, preferred_element_type=jnp.float32)
    xsq = jnp.sum(xb * xb, axis=0)
    xsq_i = jnp.sum(xi * xi, axis=0)
    d = 2.0 * inner - xsq[None, :] - xsq_i[:, None]

    nbits = (n - 1).bit_length()
    lowmask = jnp.int32((1 << nbits) - 1)
    rowmin = jnp.min(d, axis=1, keepdims=True)
    scale = (2.0 ** 20 - 2.0) / jnp.maximum(-rowmin, 1e-30)
    qd = lax.convert_element_type(d * scale, jnp.int32)
    cols = lax.broadcasted_iota(jnp.int32, (rows, n), 1)
    key = (qd * jnp.int32(1 << nbits)) | (jnp.int32(n - 1) - cols)

    neg_inf = jnp.int32(-(2 ** 31))
    base = pl.program_id(0) * n
    picks = []
    for _ in range(K):
        m = jnp.max(key, axis=1, keepdims=True)
        picks.append(jnp.int32(n - 1) - (m & lowmask) + base)
        key = jnp.where(key == m, neg_inf, key)
    idx_ref[0] = jnp.concatenate(picks, axis=1)


def _mlp_body(g_ref, bv_ref, w2t_ref, out_ref):
    h = g_ref[0] + bv_ref[...]
    h = jnp.where(h > 0, h, 0.2 * h)
    h2 = jnp.dot(h, w2t_ref[...], preferred_element_type=jnp.float32)
    h2 = jnp.where(h2 > 0, h2, 0.2 * h2)

    @pl.when(pl.program_id(1) == 0)
    def _():
        out_ref[...] = h2

    @pl.when(pl.program_id(1) != 0)
    def _():
        out_ref[...] = jnp.maximum(out_ref[...], h2)


def _sc_gather(table, idx):
    num = idx.shape[0]
    d_dim = table.shape[1]
    idx2 = idx.reshape(1, num)
    mesh = plsc.VectorSubcoreMesh(core_axis_name="c", subcore_axis_name="s")

    @functools.partial(
        pl.kernel,
        out_type=jax.ShapeDtypeStruct((num, d_dim), table.dtype),
        mesh=mesh,
    )
    def gk(table_hbm, idx_hbm, out_hbm):
        def body(i_vmem, o_vmem):
            pltpu.sync_copy(table_hbm.at[i_vmem.at[0]], o_vmem)

        pltpu.emit_pipeline(
            body,
            grid=(num // GW,),
            in_specs=[pl.BlockSpec((1, GW), lambda i: (0, i))],
            out_specs=[pl.BlockSpec((GW, d_dim), lambda i: (i, 0))],
            core_axis_name=("c", "s"),
            dimension_semantics=(pltpu.PARALLEL,),
        )(idx_hbm, out_hbm)

    return gk(table, idx2)


def kernel(x, W1, W2):
    B, C, N = x.shape
    O1 = W1.shape[0]
    O2 = W2.shape[0]
    w1at = jnp.transpose(W1[:, :C])
    wdt = jnp.transpose(W1[:, C:] - W1[:, :C])
    w2t = jnp.transpose(W2)

    idxg, a_rows, b_rows = pl.pallas_call(
        _knn_body,
        grid=(B, N // ROWS),
        in_specs=[
            pl.BlockSpec((1, C, N), lambda b, i: (b, 0, 0)),
            pl.BlockSpec((1, C, ROWS), lambda b, i: (b, 0, i)),
            pl.BlockSpec((C, O1), lambda b, i: (0, 0)),
            pl.BlockSpec((C, O1), lambda b, i: (0, 0)),
        ],
        out_specs=[
            pl.BlockSpec((1, ROWS, K), lambda b, i: (b, i, 0)),
            pl.BlockSpec((1, ROWS, O1), lambda b, i: (b, i, 0)),
            pl.BlockSpec((1, ROWS, O1), lambda b, i: (b, i, 0)),
        ],
        out_shape=[
            jax.ShapeDtypeStruct((B, N, K), jnp.int32),
            jax.ShapeDtypeStruct((B, N, O1), jnp.float32),
            jax.ShapeDtypeStruct((B, N, O1), jnp.float32),
        ],
    )(x, x, w1at, wdt)

    idx_flat = jnp.transpose(idxg.reshape(B * N, K)).reshape(-1)
    gathered = _sc_gather(a_rows.reshape(B * N, O1), idx_flat)

    out = pl.pallas_call(
        _mlp_body,
        grid=(B * N // R3, K),
        in_specs=[
            pl.BlockSpec((1, R3, O1), lambda i, j: (j, i, 0)),
            pl.BlockSpec((R3, O1), lambda i, j: (i, 0)),
            pl.BlockSpec((O1, O2), lambda i, j: (0, 0)),
        ],
        out_specs=pl.BlockSpec((R3, O2), lambda i, j: (i, 0)),
        out_shape=jax.ShapeDtypeStruct((B * N, O2), jnp.float32),
    )(gathered.reshape(K, B * N, O1), b_rows.reshape(B * N, O1), w2t)

    return jnp.swapaxes(out.reshape(B, N, O2), 1, 2)
